# baseline (speedup 1.0000x reference)
import functools

import jax
import jax.numpy as jnp
from jax.experimental import pallas as pl


def _dot(a, b):
    return jnp.dot(a, b, preferred_element_type=jnp.float32)


def _fused_kernel(
    adj1_r, adj2_r, opma_r, opmaT_r, e0_r, e1_r, opes_r, mas_r, glo_r,
    jobs_r, elig_r, act_r,
    opW1_0_r, opb1_0_r, opW2_0_r, opb2_0_r, maW_0_r, mab_0_r,
    opW1_1_r, opb1_1_r, opW2_1_r, opb2_1_r, maW_1_r, mab_1_r,
    gW1_r, gb1_r, gW2_r, gb2_r,
    aW0_r, ab0_r, aW1_r, ab1_r, aW2_r, ab2_r,
    cW0_r, cb0_r, cW1_r, cb1_r, cW2_r, cb2_r,
    out_r,
    *, O, M, J, OUT):
    def elu(t):
        return jnp.where(t > 0.0, t, jnp.exp(jnp.minimum(t, 0.0)) - 1.0)

    bf16 = jnp.bfloat16
    adj1 = adj1_r[0].astype(bf16)
    adj2 = adj2_r[0].astype(bf16)
    opma = opma_r[0]
    opmaT = opmaT_r[0]
    opma16 = opma.astype(bf16)
    opmaT16 = opmaT.astype(bf16)
    e0 = e0_r[0]
    e1 = e1_r[0]
    x = opes_r[0]
    xm = mas_r[0]

    agg_edge = jnp.concatenate(
        [jnp.sum(opma * e0, axis=1, keepdims=True),
         jnp.sum(opma * e1, axis=1, keepdims=True)], axis=1)
    P0 = _dot(opmaT, e0)
    P1 = _dot(opmaT, e1)
    eye = (jax.lax.broadcasted_iota(jnp.int32, (M, M), 0)
           == jax.lax.broadcasted_iota(jnp.int32, (M, M), 1))
    d0 = jnp.sum(jnp.where(eye, P0, 0.0), axis=1, keepdims=True)
    d1 = jnp.sum(jnp.where(eye, P1, 0.0), axis=1, keepdims=True)
    agg_e2 = jnp.concatenate([d0, d1], axis=1)

    layer_params = [
        (opW1_0_r, opb1_0_r, opW2_0_r, opb2_0_r, maW_0_r, mab_0_r),
        (opW1_1_r, opb1_1_r, opW2_1_r, opb2_1_r, maW_1_r, mab_1_r),
    ]
    for (W1_r, b1_r, W2_r, b2_r, mW_r, mb_r) in layer_params:
        x_hi = x.astype(bf16)
        x_lo = (x - x_hi.astype(jnp.float32)).astype(bf16)
        xm16 = xm.astype(bf16)
        pre = _dot(adj1, x_hi) + _dot(adj1, x_lo)
        solved = _dot(adj2, x_hi) + _dot(adj2, x_lo)
        aggma = _dot(opma16, xm16)
        h = jnp.concatenate([x, pre, solved, aggma, agg_edge], axis=1)
        h = elu(_dot(h.astype(bf16), W1_r[...]) + b1_r[...])
        x = elu(_dot(h.astype(bf16), W2_r[...]) + b2_r[...])
        agg_ope = _dot(opmaT16, x.astype(bf16))
        hm = jnp.concatenate([xm, agg_ope, agg_e2], axis=1)
        xm = elu(_dot(hm.astype(bf16), mW_r[...]) + mb_r[...])

    pooled = jnp.concatenate([jnp.mean(x, axis=0, keepdims=True),
                              jnp.mean(xm, axis=0, keepdims=True)], axis=1)
    g = elu(_dot(glo_r[0], gW1_r[...]) + gb1_r[...])
    g = elu(_dot(g, gW2_r[...]) + gb2_r[...])

    jobs = jobs_r[0]
    onehot = (jobs == jax.lax.broadcasted_iota(jnp.int32, (J, O), 1)
              ).astype(jnp.float32)
    hj = _dot(onehot, x)

    aW0 = aW0_r[...]
    u = _dot(hj, aW0[:OUT])
    v = _dot(xm, aW0[OUT:])
    w = _dot(g - pooled, aW0) + ab0_r[...]
    t0 = jnp.tanh(u[:, None, :] + (v + w)[None, :, :])
    t0 = t0.reshape(J * M, t0.shape[-1])
    t1 = jnp.tanh(_dot(t0, aW1_r[...]) + ab1_r[...])
    s = jnp.sum(t1 * aW2_r[...], axis=1, keepdims=True) + ab2_r[...]

    mask = elig_r[0]
    s = jnp.where(mask > 0.0, s, -1e9)
    mx = jnp.max(s, axis=0, keepdims=True)
    z = s - mx
    ex = jnp.exp(z)
    Z = jnp.sum(ex, axis=0, keepdims=True)
    logp = z - jnp.log(Z)
    probs = ex / Z
    ent = -jnp.sum(probs * logp, axis=0, keepdims=True)
    a = act_r[0]
    ridx = jax.lax.broadcasted_iota(jnp.int32, (J * M, 1), 0)
    alp = jnp.sum(jnp.where(ridx == a, logp, 0.0), axis=0, keepdims=True)

    tc = jnp.tanh(_dot(pooled + g, cW0_r[...]) + cb0_r[...])
    tc = jnp.tanh(_dot(tc, cW1_r[...]) + cb1_r[...])
    sv = jnp.sum(tc * cW2_r[...], axis=1, keepdims=True) + cb2_r[...]

    lane = jax.lax.broadcasted_iota(jnp.int32, (1, 128), 1)
    vec = (jnp.where(lane == 0, alp, 0.0)
           + jnp.where(lane == 1, sv, 0.0)
           + jnp.where(lane == 2, ent, 0.0))
    out_r[0] = vec


def kernel(raw_opes, raw_mas, raw_edge, op_adj_in, ma_adj_in, op_ma_adj,
           norm_glo, params, jobs_gather, eligible, action_envs):
    B, O, _ = raw_opes.shape
    M = raw_mas.shape[1]
    J = jobs_gather.shape[1]
    OUT = params['opW2_0'].shape[1]

    e0 = raw_edge[..., 0]
    e1 = raw_edge[..., 1]
    opmaT = jnp.swapaxes(op_ma_adj, 1, 2)
    jobs3 = jobs_gather.astype(jnp.int32).reshape(B, J, 1)
    act3 = action_envs.astype(jnp.int32).reshape(B, 1, 1)
    elig = eligible.astype(jnp.float32).reshape(B, J * M, 1)
    glo3 = norm_glo.reshape(B, 1, norm_glo.shape[1])

    p = params
    row = lambda t: t.reshape(1, -1)
    bf = lambda t: t.astype(jnp.bfloat16)
    weights = [
        bf(p['opW1_0']), row(p['opb1_0']), bf(p['opW2_0']), row(p['opb2_0']),
        bf(p['maW_0']), row(p['mab_0']),
        bf(p['opW1_1']), row(p['opb1_1']), bf(p['opW2_1']), row(p['opb2_1']),
        bf(p['maW_1']), row(p['mab_1']),
        p['gW1'], row(p['gb1']), p['gW2'], row(p['gb2']),
        p['aW0'], row(p['ab0']), p['aW1'], row(p['ab1']),
        row(p['aW2']), row(p['ab2']),
        p['cW0'], row(p['cb0']), p['cW1'], row(p['cb1']),
        row(p['cW2']), row(p['cb2']),
    ]

    def bspec(shape):
        nd = len(shape)
        return pl.BlockSpec((1,) + tuple(shape[1:]),
                            lambda b, _nd=nd: (b,) + (0,) * (_nd - 1))

    def wspec(shape):
        nd = len(shape)
        return pl.BlockSpec(tuple(shape), lambda b, _nd=nd: (0,) * _nd)

    batched = [op_adj_in, ma_adj_in, op_ma_adj, opmaT, e0, e1,
               raw_opes, raw_mas, glo3, jobs3, elig, act3]
    in_specs = [bspec(t.shape) for t in batched] + [wspec(t.shape) for t in weights]

    out = pl.pallas_call(
        functools.partial(_fused_kernel, O=O, M=M, J=J, OUT=OUT),
        grid=(B,),
        in_specs=in_specs,
        out_specs=pl.BlockSpec((1, 1, 128), lambda b: (b, 0, 0)),
        out_shape=jax.ShapeDtypeStruct((B, 1, 128), jnp.float32),
    )(*batched, *weights)

    return out[:, 0, :3]


# speedup vs baseline: 1.1528x; 1.1528x over previous
import functools

import jax
import jax.numpy as jnp
from jax.experimental import pallas as pl


def _dot(a, b):
    return jnp.dot(a, b, preferred_element_type=jnp.float32)


def _fused_kernel(
    adj1_r, adj2_r, opma_r, opmaT_r, e0_r, e1_r, opes_r, mas_r, glo_r,
    jobs_r, elig_r, act_r,
    opW1_0_r, opb1_0_r, opW2_0_r, opb2_0_r, maW_0_r, mab_0_r,
    opW1_1_r, opb1_1_r, opW2_1_r, opb2_1_r, maW_1_r, mab_1_r,
    gW1_r, gb1_r, gW2_r, gb2_r,
    aW0_r, ab0_r, aW1_r, ab1_r, aW2_r, ab2_r,
    cW0_r, cb0_r, cW1_r, cb1_r, cW2_r, cb2_r,
    out_r,
    *, O, M, J, OUT):
    def elu(t):
        return jnp.where(t > 0.0, t, jnp.exp(jnp.minimum(t, 0.0)) - 1.0)

    bf16 = jnp.bfloat16
    adj1 = adj1_r[0].astype(bf16)
    adj2 = adj2_r[0].astype(bf16)
    opma = opma_r[0]
    opmaT = opmaT_r[0]
    opma16 = opma.astype(bf16)
    opmaT16 = opmaT.astype(bf16)
    e0 = e0_r[0]
    e1 = e1_r[0]
    x = opes_r[0]
    xm = mas_r[0]

    agg_edge = jnp.concatenate(
        [jnp.sum(opma * e0, axis=1, keepdims=True),
         jnp.sum(opma * e1, axis=1, keepdims=True)], axis=1)
    P0 = _dot(opmaT, e0)
    P1 = _dot(opmaT, e1)
    eye = (jax.lax.broadcasted_iota(jnp.int32, (M, M), 0)
           == jax.lax.broadcasted_iota(jnp.int32, (M, M), 1))
    d0 = jnp.sum(jnp.where(eye, P0, 0.0), axis=1, keepdims=True)
    d1 = jnp.sum(jnp.where(eye, P1, 0.0), axis=1, keepdims=True)
    agg_e2 = jnp.concatenate([d0, d1], axis=1)

    layer_params = [
        (opW1_0_r, opb1_0_r, opW2_0_r, opb2_0_r, maW_0_r, mab_0_r),
        (opW1_1_r, opb1_1_r, opW2_1_r, opb2_1_r, maW_1_r, mab_1_r),
    ]
    for (W1_r, b1_r, W2_r, b2_r, mW_r, mb_r) in layer_params:
        x16 = x.astype(bf16)
        xm16 = xm.astype(bf16)
        pre = _dot(adj1, x16)
        solved = _dot(adj2, x16)
        aggma = _dot(opma16, xm16)
        h = jnp.concatenate([x, pre, solved, aggma, agg_edge], axis=1)
        h = elu(_dot(h.astype(bf16), W1_r[...]) + b1_r[...])
        x = elu(_dot(h.astype(bf16), W2_r[...]) + b2_r[...])
        agg_ope = _dot(opmaT16, x.astype(bf16))
        hm = jnp.concatenate([xm, agg_ope, agg_e2], axis=1)
        xm = elu(_dot(hm.astype(bf16), mW_r[...]) + mb_r[...])

    pooled = jnp.concatenate([jnp.mean(x, axis=0, keepdims=True),
                              jnp.mean(xm, axis=0, keepdims=True)], axis=1)
    g = elu(_dot(glo_r[0], gW1_r[...]) + gb1_r[...])
    g = elu(_dot(g, gW2_r[...]) + gb2_r[...])

    jobs = jobs_r[0]
    onehot = (jobs == jax.lax.broadcasted_iota(jnp.int32, (J, O), 1)
              ).astype(jnp.float32)
    hj = _dot(onehot, x)

    aW0 = aW0_r[...]
    u = _dot(hj, aW0[:OUT])
    v = _dot(xm, aW0[OUT:])
    w = _dot(g - pooled, aW0) + ab0_r[...]
    t0 = jnp.tanh(u[:, None, :] + (v + w)[None, :, :])
    t0 = t0.reshape(J * M, t0.shape[-1])
    t1 = jnp.tanh(_dot(t0, aW1_r[...]) + ab1_r[...])
    s = jnp.sum(t1 * aW2_r[...], axis=1, keepdims=True) + ab2_r[...]

    mask = elig_r[0]
    s = jnp.where(mask > 0.0, s, -1e9)
    mx = jnp.max(s, axis=0, keepdims=True)
    z = s - mx
    ex = jnp.exp(z)
    Z = jnp.sum(ex, axis=0, keepdims=True)
    logp = z - jnp.log(Z)
    probs = ex / Z
    ent = -jnp.sum(probs * logp, axis=0, keepdims=True)
    a = act_r[0]
    ridx = jax.lax.broadcasted_iota(jnp.int32, (J * M, 1), 0)
    alp = jnp.sum(jnp.where(ridx == a, logp, 0.0), axis=0, keepdims=True)

    tc = jnp.tanh(_dot(pooled + g, cW0_r[...]) + cb0_r[...])
    tc = jnp.tanh(_dot(tc, cW1_r[...]) + cb1_r[...])
    sv = jnp.sum(tc * cW2_r[...], axis=1, keepdims=True) + cb2_r[...]

    lane = jax.lax.broadcasted_iota(jnp.int32, (1, 128), 1)
    vec = (jnp.where(lane == 0, alp, 0.0)
           + jnp.where(lane == 1, sv, 0.0)
           + jnp.where(lane == 2, ent, 0.0))
    out_r[0] = vec


def kernel(raw_opes, raw_mas, raw_edge, op_adj_in, ma_adj_in, op_ma_adj,
           norm_glo, params, jobs_gather, eligible, action_envs):
    B, O, _ = raw_opes.shape
    M = raw_mas.shape[1]
    J = jobs_gather.shape[1]
    OUT = params['opW2_0'].shape[1]

    e0 = raw_edge[..., 0]
    e1 = raw_edge[..., 1]
    opmaT = jnp.swapaxes(op_ma_adj, 1, 2)
    jobs3 = jobs_gather.astype(jnp.int32).reshape(B, J, 1)
    act3 = action_envs.astype(jnp.int32).reshape(B, 1, 1)
    elig = eligible.astype(jnp.float32).reshape(B, J * M, 1)
    glo3 = norm_glo.reshape(B, 1, norm_glo.shape[1])

    p = params
    row = lambda t: t.reshape(1, -1)
    bf = lambda t: t.astype(jnp.bfloat16)
    weights = [
        bf(p['opW1_0']), row(p['opb1_0']), bf(p['opW2_0']), row(p['opb2_0']),
        bf(p['maW_0']), row(p['mab_0']),
        bf(p['opW1_1']), row(p['opb1_1']), bf(p['opW2_1']), row(p['opb2_1']),
        bf(p['maW_1']), row(p['mab_1']),
        p['gW1'], row(p['gb1']), p['gW2'], row(p['gb2']),
        p['aW0'], row(p['ab0']), p['aW1'], row(p['ab1']),
        row(p['aW2']), row(p['ab2']),
        p['cW0'], row(p['cb0']), p['cW1'], row(p['cb1']),
        row(p['cW2']), row(p['cb2']),
    ]

    def bspec(shape):
        nd = len(shape)
        return pl.BlockSpec((1,) + tuple(shape[1:]),
                            lambda b, _nd=nd: (b,) + (0,) * (_nd - 1))

    def wspec(shape):
        nd = len(shape)
        return pl.BlockSpec(tuple(shape), lambda b, _nd=nd: (0,) * _nd)

    batched = [op_adj_in, ma_adj_in, op_ma_adj, opmaT, e0, e1,
               raw_opes, raw_mas, glo3, jobs3, elig, act3]
    in_specs = [bspec(t.shape) for t in batched] + [wspec(t.shape) for t in weights]

    out = pl.pallas_call(
        functools.partial(_fused_kernel, O=O, M=M, J=J, OUT=OUT),
        grid=(B,),
        in_specs=in_specs,
        out_specs=pl.BlockSpec((1, 1, 128), lambda b: (b, 0, 0)),
        out_shape=jax.ShapeDtypeStruct((B, 1, 128), jnp.float32),
    )(*batched, *weights)

    return out[:, 0, :3]


# speedup vs baseline: 1.2022x; 1.0429x over previous
import functools

import jax
import jax.numpy as jnp
from jax.experimental import pallas as pl


def _dot(a, b):
    return jnp.dot(a, b, preferred_element_type=jnp.float32)


def _fused_kernel(
    adj1_r, adj2_r, opma_r, opmaT_r, e0_r, e1_r, opes_r, mas_r, glo_r,
    jobs_r, elig_r, act_r,
    opW1_0_r, opb1_0_r, opW2_0_r, opb2_0_r, maW_0_r, mab_0_r,
    opW1_1_r, opb1_1_r, opW2_1_r, opb2_1_r, maW_1_r, mab_1_r,
    gW1_r, gb1_r, gW2_r, gb2_r,
    aW0_r, ab0_r, aW1_r, ab1_r, aW2_r, ab2_r,
    cW0_r, cb0_r, cW1_r, cb1_r, cW2_r, cb2_r,
    out_r,
    *, O, M, J, OUT, EPS):
    def elu(t):
        return jnp.where(t > 0.0, t, jnp.exp(jnp.minimum(t, 0.0)) - 1.0)

    bf16 = jnp.bfloat16
    for i in range(EPS):
        _one_env(
            i, elu, bf16,
            adj1_r, adj2_r, opma_r, opmaT_r, e0_r, e1_r, opes_r, mas_r,
            glo_r, jobs_r, elig_r, act_r,
            opW1_0_r, opb1_0_r, opW2_0_r, opb2_0_r, maW_0_r, mab_0_r,
            opW1_1_r, opb1_1_r, opW2_1_r, opb2_1_r, maW_1_r, mab_1_r,
            gW1_r, gb1_r, gW2_r, gb2_r,
            aW0_r, ab0_r, aW1_r, ab1_r, aW2_r, ab2_r,
            cW0_r, cb0_r, cW1_r, cb1_r, cW2_r, cb2_r,
            out_r, O, M, J, OUT)


def _one_env(
    i, elu, bf16,
    adj1_r, adj2_r, opma_r, opmaT_r, e0_r, e1_r, opes_r, mas_r, glo_r,
    jobs_r, elig_r, act_r,
    opW1_0_r, opb1_0_r, opW2_0_r, opb2_0_r, maW_0_r, mab_0_r,
    opW1_1_r, opb1_1_r, opW2_1_r, opb2_1_r, maW_1_r, mab_1_r,
    gW1_r, gb1_r, gW2_r, gb2_r,
    aW0_r, ab0_r, aW1_r, ab1_r, aW2_r, ab2_r,
    cW0_r, cb0_r, cW1_r, cb1_r, cW2_r, cb2_r,
    out_r, O, M, J, OUT):
    adj1 = adj1_r[i].astype(bf16)
    adj2 = adj2_r[i].astype(bf16)
    opma = opma_r[i]
    opmaT = opmaT_r[i]
    opma16 = opma.astype(bf16)
    opmaT16 = opmaT.astype(bf16)
    e0 = e0_r[i]
    e1 = e1_r[i]
    x = opes_r[i]
    xm = mas_r[i]

    agg_edge = jnp.concatenate(
        [jnp.sum(opma * e0, axis=1, keepdims=True),
         jnp.sum(opma * e1, axis=1, keepdims=True)], axis=1)
    P0 = _dot(opmaT, e0)
    P1 = _dot(opmaT, e1)
    eye = (jax.lax.broadcasted_iota(jnp.int32, (M, M), 0)
           == jax.lax.broadcasted_iota(jnp.int32, (M, M), 1))
    d0 = jnp.sum(jnp.where(eye, P0, 0.0), axis=1, keepdims=True)
    d1 = jnp.sum(jnp.where(eye, P1, 0.0), axis=1, keepdims=True)
    agg_e2 = jnp.concatenate([d0, d1], axis=1)

    layer_params = [
        (opW1_0_r, opb1_0_r, opW2_0_r, opb2_0_r, maW_0_r, mab_0_r),
        (opW1_1_r, opb1_1_r, opW2_1_r, opb2_1_r, maW_1_r, mab_1_r),
    ]
    for (W1_r, b1_r, W2_r, b2_r, mW_r, mb_r) in layer_params:
        x16 = x.astype(bf16)
        xm16 = xm.astype(bf16)
        pre = _dot(adj1, x16)
        solved = _dot(adj2, x16)
        aggma = _dot(opma16, xm16)
        h = jnp.concatenate([x, pre, solved, aggma, agg_edge], axis=1)
        h = elu(_dot(h.astype(bf16), W1_r[...]) + b1_r[...])
        x = elu(_dot(h.astype(bf16), W2_r[...]) + b2_r[...])
        agg_ope = _dot(opmaT16, x.astype(bf16))
        hm = jnp.concatenate([xm, agg_ope, agg_e2], axis=1)
        xm = elu(_dot(hm.astype(bf16), mW_r[...]) + mb_r[...])

    pooled = jnp.concatenate([jnp.mean(x, axis=0, keepdims=True),
                              jnp.mean(xm, axis=0, keepdims=True)], axis=1)
    g = elu(_dot(glo_r[i], gW1_r[...]) + gb1_r[...])
    g = elu(_dot(g, gW2_r[...]) + gb2_r[...])

    jobs = jobs_r[i]
    onehot = (jobs == jax.lax.broadcasted_iota(jnp.int32, (J, O), 1)
              ).astype(jnp.float32)
    hj = _dot(onehot, x)

    aW0 = aW0_r[...]
    u = _dot(hj, aW0[:OUT])
    v = _dot(xm, aW0[OUT:])
    w = _dot(g - pooled, aW0) + ab0_r[...]
    t0 = jnp.tanh(u[:, None, :] + (v + w)[None, :, :])
    t0 = t0.reshape(J * M, t0.shape[-1])
    t1 = jnp.tanh(_dot(t0, aW1_r[...]) + ab1_r[...])
    s = jnp.sum(t1 * aW2_r[...], axis=1, keepdims=True) + ab2_r[...]

    mask = elig_r[i]
    s = jnp.where(mask > 0.0, s, -1e9)
    mx = jnp.max(s, axis=0, keepdims=True)
    z = s - mx
    ex = jnp.exp(z)
    Z = jnp.sum(ex, axis=0, keepdims=True)
    logp = z - jnp.log(Z)
    probs = ex / Z
    ent = -jnp.sum(probs * logp, axis=0, keepdims=True)
    a = act_r[i]
    ridx = jax.lax.broadcasted_iota(jnp.int32, (J * M, 1), 0)
    alp = jnp.sum(jnp.where(ridx == a, logp, 0.0), axis=0, keepdims=True)

    tc = jnp.tanh(_dot(pooled + g, cW0_r[...]) + cb0_r[...])
    tc = jnp.tanh(_dot(tc, cW1_r[...]) + cb1_r[...])
    sv = jnp.sum(tc * cW2_r[...], axis=1, keepdims=True) + cb2_r[...]

    lane = jax.lax.broadcasted_iota(jnp.int32, (1, 128), 1)
    vec = (jnp.where(lane == 0, alp, 0.0)
           + jnp.where(lane == 1, sv, 0.0)
           + jnp.where(lane == 2, ent, 0.0))
    out_r[i] = vec


def kernel(raw_opes, raw_mas, raw_edge, op_adj_in, ma_adj_in, op_ma_adj,
           norm_glo, params, jobs_gather, eligible, action_envs):
    B, O, _ = raw_opes.shape
    M = raw_mas.shape[1]
    J = jobs_gather.shape[1]
    OUT = params['opW2_0'].shape[1]

    e0 = raw_edge[..., 0]
    e1 = raw_edge[..., 1]
    opmaT = jnp.swapaxes(op_ma_adj, 1, 2)
    jobs3 = jobs_gather.astype(jnp.int32).reshape(B, J, 1)
    act3 = action_envs.astype(jnp.int32).reshape(B, 1, 1)
    elig = eligible.astype(jnp.float32).reshape(B, J * M, 1)
    glo3 = norm_glo.reshape(B, 1, norm_glo.shape[1])

    p = params
    row = lambda t: t.reshape(1, -1)
    bf = lambda t: t.astype(jnp.bfloat16)
    weights = [
        bf(p['opW1_0']), row(p['opb1_0']), bf(p['opW2_0']), row(p['opb2_0']),
        bf(p['maW_0']), row(p['mab_0']),
        bf(p['opW1_1']), row(p['opb1_1']), bf(p['opW2_1']), row(p['opb2_1']),
        bf(p['maW_1']), row(p['mab_1']),
        p['gW1'], row(p['gb1']), p['gW2'], row(p['gb2']),
        p['aW0'], row(p['ab0']), p['aW1'], row(p['ab1']),
        row(p['aW2']), row(p['ab2']),
        p['cW0'], row(p['cb0']), p['cW1'], row(p['cb1']),
        row(p['cW2']), row(p['cb2']),
    ]

    EPS = 2

    def bspec(shape):
        nd = len(shape)
        return pl.BlockSpec((EPS,) + tuple(shape[1:]),
                            lambda b, _nd=nd: (b,) + (0,) * (_nd - 1))

    def wspec(shape):
        nd = len(shape)
        return pl.BlockSpec(tuple(shape), lambda b, _nd=nd: (0,) * _nd)

    batched = [op_adj_in, ma_adj_in, op_ma_adj, opmaT, e0, e1,
               raw_opes, raw_mas, glo3, jobs3, elig, act3]
    in_specs = [bspec(t.shape) for t in batched] + [wspec(t.shape) for t in weights]

    out = pl.pallas_call(
        functools.partial(_fused_kernel, O=O, M=M, J=J, OUT=OUT, EPS=EPS),
        grid=(B // EPS,),
        in_specs=in_specs,
        out_specs=pl.BlockSpec((EPS, 1, 128), lambda b: (b, 0, 0)),
        out_shape=jax.ShapeDtypeStruct((B, 1, 128), jnp.float32),
    )(*batched, *weights)

    return out[:, 0, :3]
